# baseline (speedup 1.0000x reference)
import dataclasses
import functools

import jax
import jax.numpy as jnp
from jax import lax
from jax.experimental import pallas as pl
from jax.experimental.pallas import tpu as pltpu
from jax.experimental.pallas import tpu_sc as plsc

N_NODES = 10000
NP = 10240
D = 128
F = 128
NEG = 0.2
NCORES = 2
NSUB = 16
NW = NCORES * NSUB
CHUNK = 96
LANES = 16
TC_BLK = 1024


def _tc_project(x_p, W, attn_l, attn_r):
    def body(x_ref, w_ref, al_ref, ar_ref, h_ref, el_ref, er_ref):
        xb = x_ref[...]
        h = lax.dot_general(xb, w_ref[...], (((1,), (1,)), ((), ())),
                            preferred_element_type=jnp.float32)
        h_ref[...] = h
        el_ref[...] = (h * al_ref[0]).sum(axis=1, keepdims=True)
        er_ref[...] = (h * ar_ref[0]).sum(axis=1, keepdims=True)

    return pl.pallas_call(
        body,
        grid=(NP // TC_BLK,),
        in_specs=[
            pl.BlockSpec((TC_BLK, D), lambda i: (i, 0)),
            pl.BlockSpec((F, D), lambda i: (0, 0)),
            pl.BlockSpec((1, 1, F), lambda i: (0, 0, 0)),
            pl.BlockSpec((1, 1, F), lambda i: (0, 0, 0)),
        ],
        out_specs=[
            pl.BlockSpec((TC_BLK, F), lambda i: (i, 0)),
            pl.BlockSpec((TC_BLK, 1), lambda i: (i, 0)),
            pl.BlockSpec((TC_BLK, 1), lambda i: (i, 0)),
        ],
        out_shape=[
            jax.ShapeDtypeStruct((NP, F), jnp.float32),
            jax.ShapeDtypeStruct((NP, 1), jnp.float32),
            jax.ShapeDtypeStruct((NP, 1), jnp.float32),
        ],
    )(x_p, W, attn_l, attn_r)


SPLIT0 = 0.67


def _sc_edge(h_p, el, er, src3, dst3, n0, n1):
    rows_per = NP // NSUB
    mesh = plsc.VectorSubcoreMesh(core_axis_name="c", subcore_axis_name="s")
    cp = pltpu.CompilerParams()
    if "needs_layout_passes" in pltpu.CompilerParams.__dataclass_fields__:
        cp = dataclasses.replace(cp, needs_layout_passes=False)

    @functools.partial(
        pl.kernel,
        out_type=[
            jax.ShapeDtypeStruct((NCORES, NP, F), jnp.float32),
            jax.ShapeDtypeStruct((NCORES, NP), jnp.float32),
        ],
        mesh=mesh,
        compiler_params=cp,
        scratch_types=[
            pltpu.VMEM((NP,), jnp.float32),
            pltpu.VMEM((NP,), jnp.float32),
            pltpu.VMEM((2, CHUNK), jnp.int32),
            pltpu.VMEM((2, CHUNK), jnp.int32),
            pltpu.VMEM((2, CHUNK, F), jnp.float32),
            pltpu.VMEM((CHUNK,), jnp.float32),
            pltpu.VMEM_SHARED((NP, F), jnp.float32),
            pltpu.VMEM_SHARED((NP,), jnp.float32),
            pltpu.SemaphoreType.DMA,
            pltpu.SemaphoreType.DMA,
        ],
    )
    def k(h_hbm, el_hbm, er_hbm, src_hbm, dst_hbm, outp_hbm, sump_hbm,
          el_v, er_v, src_v, dst_v, rows_v, e_v, out_sh, sum_sh,
          sem_g, sem_i):
        c = lax.axis_index("c")
        s = lax.axis_index("s")
        nch = jnp.where(c == 0, n0, n1)
        cstart = jnp.where(c == 0, s * n0, NSUB * n0 + s * n1)

        pltpu.sync_copy(el_hbm, el_v)
        pltpu.sync_copy(er_hbm, er_v)

        zero16 = jnp.zeros((LANES,), jnp.float32)

        @pl.loop(0, CHUNK)
        def _(i):
            for m in range(F // LANES):
                rows_v[0, i, pl.ds(m * LANES, LANES)] = zero16

        for m in range(CHUNK // LANES):
            e_v[pl.ds(m * LANES, LANES)] = zero16
        base = s * rows_per
        for b in range(rows_per // 64):
            pltpu.sync_copy(rows_v.at[0, pl.ds(0, 64), :],
                            out_sh.at[pl.ds(base + b * 64, 64), :])
            pltpu.sync_copy(e_v.at[pl.ds(0, 64)],
                            sum_sh.at[pl.ds(base + b * 64, 64)])
        plsc.subcore_barrier()

        pltpu.sync_copy(src_hbm.at[cstart], src_v.at[0])
        pltpu.sync_copy(dst_hbm.at[cstart], dst_v.at[0])
        pltpu.async_copy(h_hbm.at[src_v.at[0]], rows_v.at[0], sem_g)
        pltpu.async_copy(src_hbm.at[cstart + 1], src_v.at[1], sem_i)
        pltpu.async_copy(dst_hbm.at[cstart + 1], dst_v.at[1], sem_i)

        @pl.loop(0, nch, step=2)
        def _(j):
            for u in range(2):
                jj = j + u
                pltpu.make_async_copy(
                    h_hbm.at[src_v.at[u]], rows_v.at[u], sem_g).wait()

                @pl.when(jj + 1 < nch)
                def _():
                    pltpu.make_async_copy(
                        src_hbm.at[cstart + jj + 1], src_v.at[1 - u], sem_i).wait()
                    pltpu.make_async_copy(
                        dst_hbm.at[cstart + jj + 1], dst_v.at[1 - u], sem_i).wait()
                    pltpu.async_copy(
                        h_hbm.at[src_v.at[1 - u]], rows_v.at[1 - u], sem_g)

                for m in range(CHUNK // LANES):
                    sl = pl.ds(m * LANES, LANES)
                    sv = src_v[u, sl]
                    dv = dst_v[u, sl]
                    x = plsc.load_gather(el_v, [sv]) + plsc.load_gather(er_v, [dv])
                    e_v[sl] = jnp.where(x > 0, x, x * NEG)

                @pl.loop(0, CHUNK)
                def _(i):
                    av = plsc.load_gather(e_v, [jnp.full((LANES,), i, jnp.int32)])
                    for m in range(F // LANES):
                        sl = pl.ds(m * LANES, LANES)
                        rows_v[u, i, sl] = rows_v[u, i, sl] * av

                pltpu.sync_copy(e_v, sum_sh.at[dst_v.at[u]], add=True)
                pltpu.sync_copy(rows_v.at[u], out_sh.at[dst_v.at[u]], add=True)

                @pl.when(jj + 2 < nch)
                def _():
                    pltpu.async_copy(src_hbm.at[cstart + jj + 2], src_v.at[u], sem_i)
                    pltpu.async_copy(dst_hbm.at[cstart + jj + 2], dst_v.at[u], sem_i)

        plsc.subcore_barrier()
        pltpu.sync_copy(out_sh.at[pl.ds(base, rows_per), :],
                        outp_hbm.at[c, pl.ds(base, rows_per), :])
        pltpu.sync_copy(sum_sh.at[pl.ds(base, rows_per)],
                        sump_hbm.at[c, pl.ds(base, rows_per)])

    return k(h_p, el, er, src3, dst3)


def _tc_combine(outp, sump):
    def body(p_ref, s_ref, o_ref):
        p = p_ref[0] + p_ref[1]
        sm = s_ref[0] + s_ref[1]
        den = jnp.where(sm == 0.0, 1.0, sm)
        o_ref[...] = p / den[:, None]

    return pl.pallas_call(
        body,
        grid=(NP // TC_BLK,),
        in_specs=[
            pl.BlockSpec((NCORES, TC_BLK, F), lambda i: (0, i, 0)),
            pl.BlockSpec((NCORES, TC_BLK), lambda i: (0, i)),
        ],
        out_specs=pl.BlockSpec((TC_BLK, F), lambda i: (i, 0)),
        out_shape=jax.ShapeDtypeStruct((NP, F), jnp.float32),
    )(outp, sump)


def kernel(in_feats, edge_index, W, attn_l, attn_r):
    n, d = in_feats.shape
    e_edges = edge_index.shape[1]
    x_p = jnp.zeros((NP, D), jnp.float32).at[:n].set(in_feats)

    pairs = -(-e_edges // (NSUB * CHUNK))
    n0 = int(round(pairs * SPLIT0))
    n0 += n0 % 2
    n1 = pairs - n0
    n1 += n1 % 2
    nchunks = NSUB * (n0 + n1)
    ep = nchunks * CHUNK
    pad = ep - e_edges
    src = edge_index[0].astype(jnp.int32)
    dst = edge_index[1].astype(jnp.int32)
    pad_idx = jnp.full((pad,), N_NODES, jnp.int32)
    src2 = jnp.concatenate([src, pad_idx]).reshape(nchunks, CHUNK)
    dst2 = jnp.concatenate([dst, pad_idx]).reshape(nchunks, CHUNK)

    h_p, el, er = _tc_project(x_p, W, attn_l, attn_r)
    outp, sump = _sc_edge(h_p, el.reshape(NP), er.reshape(NP), src2, dst2, n0, n1)
    out = _tc_combine(outp, sump)
    return out[:n].reshape(n, 1, F)


# speedup vs baseline: 1.3055x; 1.3055x over previous
import dataclasses
import functools

import jax
import jax.numpy as jnp
from jax import lax
from jax.experimental import pallas as pl
from jax.experimental.pallas import tpu as pltpu
from jax.experimental.pallas import tpu_sc as plsc

N_NODES = 10000
NP = 10240
D = 128
F = 128
NEG = 0.2
NCORES = 2
NSUB = 16
NW = NCORES * NSUB
CHUNK = 96
LANES = 16
TC_BLK = 1024


def _tc_project(x_p, W, attn_l, attn_r):
    def body(x_ref, w_ref, al_ref, ar_ref, h_ref, el_ref, er_ref):
        xb = x_ref[...]
        h = lax.dot_general(xb, w_ref[...], (((1,), (1,)), ((), ())),
                            preferred_element_type=jnp.float32)
        h_ref[...] = h
        el_ref[...] = (h * al_ref[0]).sum(axis=1, keepdims=True)
        er_ref[...] = (h * ar_ref[0]).sum(axis=1, keepdims=True)

    return pl.pallas_call(
        body,
        grid=(NP // TC_BLK,),
        in_specs=[
            pl.BlockSpec((TC_BLK, D), lambda i: (i, 0)),
            pl.BlockSpec((F, D), lambda i: (0, 0)),
            pl.BlockSpec((1, 1, F), lambda i: (0, 0, 0)),
            pl.BlockSpec((1, 1, F), lambda i: (0, 0, 0)),
        ],
        out_specs=[
            pl.BlockSpec((TC_BLK, F), lambda i: (i, 0)),
            pl.BlockSpec((TC_BLK, 1), lambda i: (i, 0)),
            pl.BlockSpec((TC_BLK, 1), lambda i: (i, 0)),
        ],
        out_shape=[
            jax.ShapeDtypeStruct((NP, F), jnp.float32),
            jax.ShapeDtypeStruct((NP, 1), jnp.float32),
            jax.ShapeDtypeStruct((NP, 1), jnp.float32),
        ],
    )(x_p, W, attn_l, attn_r)


SPLIT0 = 0.64


def _sc_edge(h_p, el, er, src3, dst3, n0, n1):
    rows_per = NP // NSUB
    mesh = plsc.VectorSubcoreMesh(core_axis_name="c", subcore_axis_name="s")
    cp = pltpu.CompilerParams()
    if "needs_layout_passes" in pltpu.CompilerParams.__dataclass_fields__:
        cp = dataclasses.replace(cp, needs_layout_passes=False)

    @functools.partial(
        pl.kernel,
        out_type=[
            jax.ShapeDtypeStruct((NCORES, NP, F), jnp.float32),
            jax.ShapeDtypeStruct((NCORES, NP), jnp.float32),
        ],
        mesh=mesh,
        compiler_params=cp,
        scratch_types=[
            pltpu.VMEM((NP,), jnp.float32),
            pltpu.VMEM((NP,), jnp.float32),
            pltpu.VMEM((2, CHUNK), jnp.int32),
            pltpu.VMEM((2, CHUNK), jnp.int32),
            pltpu.VMEM((2, CHUNK, F), jnp.float32),
            pltpu.VMEM((CHUNK,), jnp.float32),
            pltpu.VMEM_SHARED((NP, F), jnp.float32),
            pltpu.VMEM_SHARED((NP,), jnp.float32),
            pltpu.SemaphoreType.DMA,
            pltpu.SemaphoreType.DMA,
        ],
    )
    def k(h_hbm, el_hbm, er_hbm, src_hbm, dst_hbm, outp_hbm, sump_hbm,
          el_v, er_v, src_v, dst_v, rows_v, e_v, out_sh, sum_sh,
          sem_g, sem_i):
        c = lax.axis_index("c")
        s = lax.axis_index("s")
        nch = jnp.where(c == 0, n0, n1)
        cstart = jnp.where(c == 0, s * n0, NSUB * n0 + s * n1)

        pltpu.sync_copy(el_hbm, el_v)
        pltpu.sync_copy(er_hbm, er_v)

        zero16 = jnp.zeros((LANES,), jnp.float32)

        @pl.loop(0, CHUNK)
        def _(i):
            for m in range(F // LANES):
                rows_v[0, i, pl.ds(m * LANES, LANES)] = zero16

        for m in range(CHUNK // LANES):
            e_v[pl.ds(m * LANES, LANES)] = zero16
        base = s * rows_per
        for b in range(rows_per // 64):
            pltpu.sync_copy(rows_v.at[0, pl.ds(0, 64), :],
                            out_sh.at[pl.ds(base + b * 64, 64), :])
            pltpu.sync_copy(e_v.at[pl.ds(0, 64)],
                            sum_sh.at[pl.ds(base + b * 64, 64)])
        plsc.subcore_barrier()

        pltpu.sync_copy(src_hbm.at[cstart], src_v.at[0])
        pltpu.sync_copy(dst_hbm.at[cstart], dst_v.at[0])
        pltpu.async_copy(h_hbm.at[src_v.at[0]], rows_v.at[0], sem_g)
        pltpu.async_copy(src_hbm.at[cstart + 1], src_v.at[1], sem_i)
        pltpu.async_copy(dst_hbm.at[cstart + 1], dst_v.at[1], sem_i)

        @pl.loop(0, nch, step=2)
        def _(j):
            for u in range(2):
                jj = j + u
                pltpu.make_async_copy(
                    h_hbm.at[src_v.at[u]], rows_v.at[u], sem_g).wait()

                @pl.when(jj + 1 < nch)
                def _():
                    pltpu.make_async_copy(
                        src_hbm.at[cstart + jj + 1], src_v.at[1 - u], sem_i).wait()
                    pltpu.make_async_copy(
                        dst_hbm.at[cstart + jj + 1], dst_v.at[1 - u], sem_i).wait()
                    pltpu.async_copy(
                        h_hbm.at[src_v.at[1 - u]], rows_v.at[1 - u], sem_g)

                for m in range(CHUNK // LANES):
                    sl = pl.ds(m * LANES, LANES)
                    sv = src_v[u, sl]
                    dv = dst_v[u, sl]
                    x = plsc.load_gather(el_v, [sv]) + plsc.load_gather(er_v, [dv])
                    e_v[sl] = jnp.where(x > 0, x, x * NEG)

                @functools.partial(plsc.parallel_loop, 0, CHUNK, unroll=2)
                def _(i):
                    av = plsc.load_gather(e_v, [jnp.full((LANES,), i, jnp.int32)])
                    for m in range(F // LANES):
                        sl = pl.ds(m * LANES, LANES)
                        rows_v[u, i, sl] = rows_v[u, i, sl] * av

                pltpu.sync_copy(e_v, sum_sh.at[dst_v.at[u]], add=True)
                pltpu.sync_copy(rows_v.at[u], out_sh.at[dst_v.at[u]], add=True)

                @pl.when(jj + 2 < nch)
                def _():
                    pltpu.async_copy(src_hbm.at[cstart + jj + 2], src_v.at[u], sem_i)
                    pltpu.async_copy(dst_hbm.at[cstart + jj + 2], dst_v.at[u], sem_i)

        plsc.subcore_barrier()
        pltpu.sync_copy(out_sh.at[pl.ds(base, rows_per), :],
                        outp_hbm.at[c, pl.ds(base, rows_per), :])
        pltpu.sync_copy(sum_sh.at[pl.ds(base, rows_per)],
                        sump_hbm.at[c, pl.ds(base, rows_per)])

    return k(h_p, el, er, src3, dst3)


def _tc_combine(outp, sump):
    def body(p_ref, s_ref, o_ref):
        p = p_ref[0] + p_ref[1]
        sm = s_ref[0] + s_ref[1]
        den = jnp.where(sm == 0.0, 1.0, sm)
        o_ref[...] = p / den[:, None]

    return pl.pallas_call(
        body,
        grid=(NP // TC_BLK,),
        in_specs=[
            pl.BlockSpec((NCORES, TC_BLK, F), lambda i: (0, i, 0)),
            pl.BlockSpec((NCORES, TC_BLK), lambda i: (0, i)),
        ],
        out_specs=pl.BlockSpec((TC_BLK, F), lambda i: (i, 0)),
        out_shape=jax.ShapeDtypeStruct((NP, F), jnp.float32),
    )(outp, sump)


def kernel(in_feats, edge_index, W, attn_l, attn_r):
    n, d = in_feats.shape
    e_edges = edge_index.shape[1]
    x_p = jnp.zeros((NP, D), jnp.float32).at[:n].set(in_feats)

    pairs = -(-e_edges // (NSUB * CHUNK))
    n0 = int(round(pairs * SPLIT0))
    n0 += n0 % 2
    n1 = pairs - n0
    n1 += n1 % 2
    nchunks = NSUB * (n0 + n1)
    ep = nchunks * CHUNK
    pad = ep - e_edges
    src = edge_index[0].astype(jnp.int32)
    dst = edge_index[1].astype(jnp.int32)
    pad_idx = jnp.full((pad,), N_NODES, jnp.int32)
    src2 = jnp.concatenate([src, pad_idx]).reshape(nchunks, CHUNK)
    dst2 = jnp.concatenate([dst, pad_idx]).reshape(nchunks, CHUNK)

    h_p, el, er = _tc_project(x_p, W, attn_l, attn_r)
    outp, sump = _sc_edge(h_p, el.reshape(NP), er.reshape(NP), src2, dst2, n0, n1)
    out = _tc_combine(outp, sump)
    return out[:n].reshape(n, 1, F)
